# baseline (speedup 1.0000x reference)
import functools

import jax
import jax.numpy as jnp
from jax import lax
from jax.experimental import pallas as pl
from jax.experimental.pallas import tpu as pltpu
from jax.experimental.pallas import tpu_sc as plsc

N_EDGES = 800000
DIM = 64
DIM_W = DIM // 2
NC = 2
NS = 16
NW = NC * NS
PER_W = N_EDGES // NW
CHUNK = 200
N_CHUNKS = PER_W // CHUNK
N_PAIRS = (N_CHUNKS - 1) // 2
N_FULL_BLOCKS = CHUNK // 16
TAIL_OFF = CHUNK - 16
SPLITS = ((0, 128), (128, CHUNK - 128)) if CHUNK > 128 else ((0, CHUNK),)

assert N_CHUNKS % 2 == 1 and CHUNK % 8 == 0 and PER_W % CHUNK == 0


def _sc_body(src_p, dst_p, src_n, dst_n, emb_user, emb_item, rel,
             out_p, out_n,
             rel_v, idx_s, idx_d, u_a, i_a, u_b, i_b, out_all, trans,
             sem_a, sem_b):
    wid = lax.axis_index("s") * NC + lax.axis_index("c")
    base = wid * PER_W

    pltpu.sync_copy(rel, rel_v)
    iota16 = lax.broadcasted_iota(jnp.int32, (16,), 0)
    rel_regs = [plsc.bitcast(rel_v[pl.ds(h * 16, 16)], jnp.bfloat16)
                for h in range(2)]

    def issue(j, u_buf, i_buf, sem):
        for lo, ln in SPLITS:
            pltpu.async_copy(emb_user.at[idx_s.at[pl.ds(j * CHUNK + lo, ln)]],
                             u_buf.at[pl.ds(lo, ln)], sem)
            pltpu.async_copy(emb_item.at[idx_d.at[pl.ds(j * CHUNK + lo, ln)]],
                             i_buf.at[pl.ds(lo, ln)], sem)

    def drain(j, u_buf, i_buf, sem):
        for lo, ln in SPLITS:
            pltpu.make_async_copy(
                emb_user.at[idx_s.at[pl.ds(j * CHUNK + lo, ln)]],
                u_buf.at[pl.ds(lo, ln)], sem).wait()
            pltpu.make_async_copy(
                emb_item.at[idx_d.at[pl.ds(j * CHUNK + lo, ln)]],
                i_buf.at[pl.ds(lo, ln)], sem).wait()

    iota17 = iota16 * 17

    def block_at(j, off, u_buf, i_buf):
        for e in range(16):
            acc = None
            for h in range(2):
                u = plsc.bitcast(u_buf[off + e, pl.ds(h * 16, 16)],
                                 jnp.bfloat16)
                iv = plsc.bitcast(i_buf[off + e, pl.ds(h * 16, 16)],
                                  jnp.bfloat16)
                p = u * (rel_regs[h] * iv)
                pa, pb = plsc.unpack(p, format=plsc.PackFormat.INTERLEAVED)
                t = pa + pb
                acc = t if acc is None else acc + t
            plsc.store_scatter(trans, [iota17 + e], acc)
        score = trans[pl.ds(0, 16)]
        for l in range(1, 16):
            score = score + trans[pl.ds(l * 17, 16)]
        out_all[pl.ds(j * CHUNK + off, 16)] = score

    def compute(j, u_buf, i_buf):
        def block_body(b, c):
            block_at(j, b * 16, u_buf, i_buf)
            return c
        lax.fori_loop(0, N_FULL_BLOCKS, block_body, 0)
        block_at(j, TAIL_OFF, u_buf, i_buf)

    for src, dst, out in ((src_p, dst_p, out_p), (src_n, dst_n, out_n)):
        pltpu.sync_copy(src.at[pl.ds(base, PER_W)], idx_s)
        pltpu.sync_copy(dst.at[pl.ds(base, PER_W)], idx_d)
        issue(0, u_a, i_a, sem_a)

        def pair_body(t, c):
            j0 = 2 * t
            issue(j0 + 1, u_b, i_b, sem_b)
            drain(j0, u_a, i_a, sem_a)
            compute(j0, u_a, i_a)
            issue(j0 + 2, u_a, i_a, sem_a)
            drain(j0 + 1, u_b, i_b, sem_b)
            compute(j0 + 1, u_b, i_b)
            return c

        lax.fori_loop(0, N_PAIRS, pair_body, 0)
        drain(N_CHUNKS - 1, u_a, i_a, sem_a)
        compute(N_CHUNKS - 1, u_a, i_a)
        pltpu.sync_copy(out_all.at[pl.ds(0, PER_W)], out.at[pl.ds(base, PER_W)])


@jax.jit
def _dist_multi(src_p, dst_p, src_n, dst_n, emb_user, emb_item, rel):
    mesh = plsc.VectorSubcoreMesh(core_axis_name="c", subcore_axis_name="s",
                                  num_cores=NC, num_subcores=NS)
    f = pl.kernel(
        _sc_body,
        out_type=(
            jax.ShapeDtypeStruct((N_EDGES,), jnp.float32),
            jax.ShapeDtypeStruct((N_EDGES,), jnp.float32),
        ),
        mesh=mesh,
        scratch_types=[
            pltpu.VMEM((DIM_W,), jnp.int32),
            pltpu.VMEM((PER_W,), jnp.int32),
            pltpu.VMEM((PER_W,), jnp.int32),
            pltpu.VMEM((CHUNK, DIM_W), jnp.int32),
            pltpu.VMEM((CHUNK, DIM_W), jnp.int32),
            pltpu.VMEM((CHUNK, DIM_W), jnp.int32),
            pltpu.VMEM((CHUNK, DIM_W), jnp.int32),
            pltpu.VMEM((PER_W,), jnp.float32),
            pltpu.VMEM((272,), jnp.float32),
            pltpu.SemaphoreType.DMA,
            pltpu.SemaphoreType.DMA,
        ],
        compiler_params=pltpu.CompilerParams(needs_layout_passes=False,
                                             use_tc_tiling_on_sc=False),
    )
    return f(src_p, dst_p, src_n, dst_n, emb_user, emb_item, rel)


def _pack_bf16(x):
    b = x.astype(jnp.bfloat16)
    return jax.lax.bitcast_convert_type(
        b.reshape(*x.shape[:-1], x.shape[-1] // 2, 2), jnp.int32)


def kernel(emb_user, emb_item, rel_embedding, edge_pos, edge_neg):
    rel = _pack_bf16(rel_embedding.reshape(DIM))
    return _dist_multi(edge_pos[0], edge_pos[1], edge_neg[0], edge_neg[1],
                       _pack_bf16(emb_user), _pack_bf16(emb_item), rel)


# speedup vs baseline: 1.1996x; 1.1996x over previous
import functools

import jax
import jax.numpy as jnp
from jax import lax
from jax.experimental import pallas as pl
from jax.experimental.pallas import tpu as pltpu
from jax.experimental.pallas import tpu_sc as plsc

N_EDGES = 800000
DIM = 64
DIM_W = DIM // 2
NC = 2
NS = 16
NW = NC * NS
PER_W = N_EDGES // NW
CHUNK = 200
N_CHUNKS = PER_W // CHUNK
N_PAIRS = (N_CHUNKS - 1) // 2
N_FULL_BLOCKS = CHUNK // 16
TAIL_OFF = CHUNK - 16
SPLITS = ((0, 128), (128, CHUNK - 128)) if CHUNK > 128 else ((0, CHUNK),)

assert N_CHUNKS % 2 == 1 and CHUNK % 8 == 0 and PER_W % CHUNK == 0


def _sc_body(src_p, dst_p, src_n, dst_n, emb_user, emb_item, rel,
             out_p, out_n,
             rel_v, idx_s, idx_d, u_a, i_a, u_b, i_b, out_all, trans,
             sem_a, sem_b):
    wid = lax.axis_index("s") * NC + lax.axis_index("c")
    base = wid * PER_W

    pltpu.sync_copy(rel, rel_v)
    iota16 = lax.broadcasted_iota(jnp.int32, (16,), 0)
    rel_regs = [rel_v[pl.ds(h * 32, 32)] for h in range(2)]

    def issue(j, u_buf, i_buf, sem):
        for lo, ln in SPLITS:
            pltpu.async_copy(emb_user.at[idx_s.at[pl.ds(j * CHUNK + lo, ln)]],
                             u_buf.at[pl.ds(lo, ln)], sem)
            pltpu.async_copy(emb_item.at[idx_d.at[pl.ds(j * CHUNK + lo, ln)]],
                             i_buf.at[pl.ds(lo, ln)], sem)

    def drain(j, u_buf, i_buf, sem):
        for lo, ln in SPLITS:
            pltpu.make_async_copy(
                emb_user.at[idx_s.at[pl.ds(j * CHUNK + lo, ln)]],
                u_buf.at[pl.ds(lo, ln)], sem).wait()
            pltpu.make_async_copy(
                emb_item.at[idx_d.at[pl.ds(j * CHUNK + lo, ln)]],
                i_buf.at[pl.ds(lo, ln)], sem).wait()

    iota17 = iota16 * 17

    def block_at(j, off, u_buf, i_buf):
        for e in range(16):
            acc = None
            for h in range(2):
                u = u_buf[off + e, pl.ds(h * 32, 32)]
                iv = i_buf[off + e, pl.ds(h * 32, 32)]
                p = u * (rel_regs[h] * iv)
                pa, pb = plsc.unpack(p, format=plsc.PackFormat.INTERLEAVED)
                t = pa + pb
                acc = t if acc is None else acc + t
            plsc.store_scatter(trans, [iota17 + e], acc)
        score = trans[pl.ds(0, 16)]
        for l in range(1, 16):
            score = score + trans[pl.ds(l * 17, 16)]
        out_all[pl.ds(j * CHUNK + off, 16)] = score

    def compute(j, u_buf, i_buf):
        def block_body(b, c):
            block_at(j, b * 16, u_buf, i_buf)
            return c
        lax.fori_loop(0, N_FULL_BLOCKS, block_body, 0)
        block_at(j, TAIL_OFF, u_buf, i_buf)

    for src, dst, out in ((src_p, dst_p, out_p), (src_n, dst_n, out_n)):
        pltpu.sync_copy(src.at[pl.ds(base, PER_W)], idx_s)
        pltpu.sync_copy(dst.at[pl.ds(base, PER_W)], idx_d)
        issue(0, u_a, i_a, sem_a)

        def pair_body(t, c):
            j0 = 2 * t
            issue(j0 + 1, u_b, i_b, sem_b)
            drain(j0, u_a, i_a, sem_a)
            compute(j0, u_a, i_a)
            issue(j0 + 2, u_a, i_a, sem_a)
            drain(j0 + 1, u_b, i_b, sem_b)
            compute(j0 + 1, u_b, i_b)
            return c

        lax.fori_loop(0, N_PAIRS, pair_body, 0)
        drain(N_CHUNKS - 1, u_a, i_a, sem_a)
        compute(N_CHUNKS - 1, u_a, i_a)
        pltpu.sync_copy(out_all.at[pl.ds(0, PER_W)], out.at[pl.ds(base, PER_W)])


@jax.jit
def _dist_multi(src_p, dst_p, src_n, dst_n, emb_user, emb_item, rel):
    mesh = plsc.VectorSubcoreMesh(core_axis_name="c", subcore_axis_name="s",
                                  num_cores=NC, num_subcores=NS)
    f = pl.kernel(
        _sc_body,
        out_type=(
            jax.ShapeDtypeStruct((N_EDGES,), jnp.float32),
            jax.ShapeDtypeStruct((N_EDGES,), jnp.float32),
        ),
        mesh=mesh,
        scratch_types=[
            pltpu.VMEM((DIM,), jnp.bfloat16),
            pltpu.VMEM((PER_W,), jnp.int32),
            pltpu.VMEM((PER_W,), jnp.int32),
            pltpu.VMEM((CHUNK, DIM), jnp.bfloat16),
            pltpu.VMEM((CHUNK, DIM), jnp.bfloat16),
            pltpu.VMEM((CHUNK, DIM), jnp.bfloat16),
            pltpu.VMEM((CHUNK, DIM), jnp.bfloat16),
            pltpu.VMEM((PER_W,), jnp.float32),
            pltpu.VMEM((272,), jnp.float32),
            pltpu.SemaphoreType.DMA,
            pltpu.SemaphoreType.DMA,
        ],
        compiler_params=pltpu.CompilerParams(needs_layout_passes=False,
                                             use_tc_tiling_on_sc=False),
    )
    return f(src_p, dst_p, src_n, dst_n, emb_user, emb_item, rel)


def kernel(emb_user, emb_item, rel_embedding, edge_pos, edge_neg):
    rel = rel_embedding.reshape(DIM).astype(jnp.bfloat16)
    return _dist_multi(edge_pos[0], edge_pos[1], edge_neg[0], edge_neg[1],
                       emb_user.astype(jnp.bfloat16),
                       emb_item.astype(jnp.bfloat16), rel)


# speedup vs baseline: 1.9235x; 1.6034x over previous
import functools

import jax
import jax.numpy as jnp
from jax import lax
from jax.experimental import pallas as pl
from jax.experimental.pallas import tpu as pltpu
from jax.experimental.pallas import tpu_sc as plsc

N_EDGES = 800000
DIM = 64
DIM_W = DIM // 2
NC = 2
NS = 16
NW = NC * NS
PER_W = N_EDGES // NW
CHUNK = 200
N_CHUNKS = PER_W // CHUNK
N_PAIRS = (N_CHUNKS - 1) // 2
N_FULL_BLOCKS = CHUNK // 16
TAIL_OFF = CHUNK - 16
SPLITS = ((0, 128), (128, CHUNK - 128)) if CHUNK > 128 else ((0, CHUNK),)

assert N_CHUNKS % 2 == 1 and CHUNK % 8 == 0 and PER_W % CHUNK == 0


def _sc_body(src_p, dst_p, src_n, dst_n, emb_user, emb_item, rel,
             out_p, out_n,
             rel_v, idx_s, idx_d, u_a, i_a, u_b, i_b, out_all, trans,
             sem_a, sem_b):
    wid = lax.axis_index("s") * NC + lax.axis_index("c")
    base = wid * PER_W

    pltpu.sync_copy(rel, rel_v)
    iota16 = lax.broadcasted_iota(jnp.int32, (16,), 0)
    rel_regs = [rel_v[pl.ds(h * 32, 32)] for h in range(2)]

    def issue(j, u_buf, i_buf, sem):
        for lo, ln in SPLITS:
            pltpu.async_copy(emb_user.at[idx_s.at[pl.ds(j * CHUNK + lo, ln)]],
                             u_buf.at[pl.ds(lo, ln)], sem)
            pltpu.async_copy(emb_item.at[idx_d.at[pl.ds(j * CHUNK + lo, ln)]],
                             i_buf.at[pl.ds(lo, ln)], sem)

    def drain(j, u_buf, i_buf, sem):
        for lo, ln in SPLITS:
            pltpu.make_async_copy(
                emb_user.at[idx_s.at[pl.ds(j * CHUNK + lo, ln)]],
                u_buf.at[pl.ds(lo, ln)], sem).wait()
            pltpu.make_async_copy(
                emb_item.at[idx_d.at[pl.ds(j * CHUNK + lo, ln)]],
                i_buf.at[pl.ds(lo, ln)], sem).wait()

    iota17 = iota16 * 17

    def block_at(j, off, u_buf, i_buf):
        G = 8
        for g in range(0, 16, G):
            us, ivs = [], []
            for e in range(g, g + G):
                us.append([u_buf[off + e, pl.ds(h * 32, 32)] for h in range(2)])
                ivs.append([i_buf[off + e, pl.ds(h * 32, 32)] for h in range(2)])
            ps = [[us[x][h] * (rel_regs[h] * ivs[x][h]) for h in range(2)]
                  for x in range(G)]
            accs = []
            for x in range(G):
                pa0, pb0 = plsc.unpack(ps[x][0],
                                       format=plsc.PackFormat.INTERLEAVED)
                pa1, pb1 = plsc.unpack(ps[x][1],
                                       format=plsc.PackFormat.INTERLEAVED)
                accs.append((pa0 + pb0) + (pa1 + pb1))
            for x in range(G):
                plsc.store_scatter(trans, [iota17 + (g + x)], accs[x])
        rows = [trans[pl.ds(l * 17, 16)] for l in range(16)]
        while len(rows) > 1:
            rows = [rows[i] + rows[i + 1] for i in range(0, len(rows), 2)]
        score = rows[0]
        out_all[pl.ds(j * CHUNK + off, 16)] = score

    def compute(j, u_buf, i_buf):
        def block_body(b, c):
            block_at(j, b * 16, u_buf, i_buf)
            return c
        lax.fori_loop(0, N_FULL_BLOCKS, block_body, 0)
        block_at(j, TAIL_OFF, u_buf, i_buf)

    for src, dst, out in ((src_p, dst_p, out_p), (src_n, dst_n, out_n)):
        pltpu.sync_copy(src.at[pl.ds(base, PER_W)], idx_s)
        pltpu.sync_copy(dst.at[pl.ds(base, PER_W)], idx_d)
        issue(0, u_a, i_a, sem_a)

        def pair_body(t, c):
            j0 = 2 * t
            issue(j0 + 1, u_b, i_b, sem_b)
            drain(j0, u_a, i_a, sem_a)
            compute(j0, u_a, i_a)
            issue(j0 + 2, u_a, i_a, sem_a)
            drain(j0 + 1, u_b, i_b, sem_b)
            compute(j0 + 1, u_b, i_b)
            return c

        lax.fori_loop(0, N_PAIRS, pair_body, 0)
        drain(N_CHUNKS - 1, u_a, i_a, sem_a)
        compute(N_CHUNKS - 1, u_a, i_a)
        pltpu.sync_copy(out_all.at[pl.ds(0, PER_W)], out.at[pl.ds(base, PER_W)])


@jax.jit
def _dist_multi(src_p, dst_p, src_n, dst_n, emb_user, emb_item, rel):
    mesh = plsc.VectorSubcoreMesh(core_axis_name="c", subcore_axis_name="s",
                                  num_cores=NC, num_subcores=NS)
    f = pl.kernel(
        _sc_body,
        out_type=(
            jax.ShapeDtypeStruct((N_EDGES,), jnp.float32),
            jax.ShapeDtypeStruct((N_EDGES,), jnp.float32),
        ),
        mesh=mesh,
        scratch_types=[
            pltpu.VMEM((DIM,), jnp.bfloat16),
            pltpu.VMEM((PER_W,), jnp.int32),
            pltpu.VMEM((PER_W,), jnp.int32),
            pltpu.VMEM((CHUNK, DIM), jnp.bfloat16),
            pltpu.VMEM((CHUNK, DIM), jnp.bfloat16),
            pltpu.VMEM((CHUNK, DIM), jnp.bfloat16),
            pltpu.VMEM((CHUNK, DIM), jnp.bfloat16),
            pltpu.VMEM((PER_W,), jnp.float32),
            pltpu.VMEM((272,), jnp.float32),
            pltpu.SemaphoreType.DMA,
            pltpu.SemaphoreType.DMA,
        ],
        compiler_params=pltpu.CompilerParams(needs_layout_passes=False,
                                             use_tc_tiling_on_sc=False),
    )
    return f(src_p, dst_p, src_n, dst_n, emb_user, emb_item, rel)


def kernel(emb_user, emb_item, rel_embedding, edge_pos, edge_neg):
    rel = rel_embedding.reshape(DIM).astype(jnp.bfloat16)
    return _dist_multi(edge_pos[0], edge_pos[1], edge_neg[0], edge_neg[1],
                       emb_user.astype(jnp.bfloat16),
                       emb_item.astype(jnp.bfloat16), rel)
